# Initial kernel scaffold; baseline (speedup 1.0000x reference)
import jax
import jax.numpy as jnp
from jax.experimental import pallas as pl


def kernel(x, edge_index, W1, b1, W2, b2, W3, b3):
    raise NotImplementedError("write your pallas kernel here")


# speedup vs baseline: 10.6528x; 10.6528x over previous
import functools

import jax
import jax.numpy as jnp
from jax import lax
from jax.experimental import pallas as pl
from jax.experimental.pallas import tpu as pltpu
from jax.experimental.pallas import tpu_sc as plsc

N_NODES = 10000
N_PAD = 10240
N_EDGES = 320000
NC = 2
NS = 16
NW = NC * NS
EPW = N_EDGES // NW
WIN = 80
NWIN = EPW // WIN
RPT = N_PAD // NS
ZROWS = 128

_MESH = plsc.VectorSubcoreMesh(core_axis_name="c", subcore_axis_name="s")


def _sc_degree(dst3):
    @functools.partial(
        pl.kernel,
        out_type=jax.ShapeDtypeStruct((NC, N_PAD), jnp.float32),
        mesh=_MESH,
        scratch_types=[
            pltpu.VMEM_SHARED((N_PAD,), jnp.float32),
            pltpu.VMEM((1, WIN), jnp.int32),
            pltpu.VMEM((1, WIN), jnp.float32),
            pltpu.VMEM((RPT,), jnp.float32),
            pltpu.SemaphoreType.DMA,
        ],
    )
    def k(dst_hbm, out_hbm, acc, idx_v, ones_v, z_v, sem):
        c = lax.axis_index("c")
        s = lax.axis_index("s")
        wid = c * NS + s

        @pl.loop(0, WIN, step=16)
        def _(i):
            ones_v[0, pl.ds(i, 16)] = jnp.ones((16,), jnp.float32)

        @pl.loop(0, RPT, step=16)
        def _(i):
            z_v[pl.ds(i, 16)] = jnp.zeros((16,), jnp.float32)

        pltpu.sync_copy(z_v, acc.at[pl.ds(s * RPT, RPT)])
        plsc.subcore_barrier()

        @pl.loop(0, NWIN)
        def _(w):
            pltpu.sync_copy(dst_hbm.at[wid, pl.ds(w, 1)], idx_v)
            pltpu.sync_copy(ones_v.at[0], acc.at[idx_v.at[0]], add=True)

        plsc.subcore_barrier()
        pltpu.sync_copy(acc.at[pl.ds(s * RPT, RPT)],
                        out_hbm.at[c, pl.ds(s * RPT, RPT)])

    return k(dst3)


def _sc_propagate(g_pad, src3, dst3, d_ch):
    @functools.partial(
        pl.kernel,
        out_type=jax.ShapeDtypeStruct((NC, N_PAD, d_ch), jnp.float32),
        mesh=_MESH,
        scratch_types=[
            pltpu.VMEM_SHARED((N_PAD, d_ch), jnp.float32),
            pltpu.VMEM((1, WIN), jnp.int32),
            pltpu.VMEM((1, WIN), jnp.int32),
            pltpu.VMEM((WIN, d_ch), jnp.float32),
            pltpu.VMEM((ZROWS, d_ch), jnp.float32),
            pltpu.SemaphoreType.DMA,
        ],
    )
    def k(g_hbm, src_hbm, dst_hbm, out_hbm, acc, sidx, didx, rows_v, z_v, sem):
        c = lax.axis_index("c")
        s = lax.axis_index("s")
        wid = c * NS + s

        @pl.loop(0, ZROWS)
        def _(r):
            @pl.loop(0, d_ch, step=16)
            def _(k16):
                z_v[r, pl.ds(k16, 16)] = jnp.zeros((16,), jnp.float32)

        @pl.loop(0, RPT, step=ZROWS)
        def _(r0):
            pltpu.sync_copy(z_v, acc.at[pl.ds(s * RPT + r0, ZROWS)])

        plsc.subcore_barrier()

        @pl.loop(0, NWIN)
        def _(w):
            pltpu.sync_copy(src_hbm.at[wid, pl.ds(w, 1)], sidx)
            pltpu.sync_copy(dst_hbm.at[wid, pl.ds(w, 1)], didx)
            pltpu.async_copy(g_hbm.at[sidx.at[0]], rows_v, sem).wait()
            pltpu.sync_copy(rows_v, acc.at[didx.at[0]], add=True)

        plsc.subcore_barrier()
        pltpu.sync_copy(acc.at[pl.ds(s * RPT, RPT)],
                        out_hbm.at[c, pl.ds(s * RPT, RPT)])

    return k(g_pad, src3, dst3)


def _tc_first(x_pad, W1, c0, c1):
    def body(x_ref, w_ref, c0_ref, c1_ref, g_ref, dinv_ref):
        dinv = lax.rsqrt(c0_ref[...] + c1_ref[...] + 1.0)
        dinv_ref[...] = dinv
        h = jnp.dot(x_ref[...], w_ref[...],
                    preferred_element_type=jnp.float32,
                    precision=lax.Precision.HIGHEST)
        g_ref[...] = h * dinv

    return pl.pallas_call(
        body,
        out_shape=(jax.ShapeDtypeStruct((N_PAD, W1.shape[1]), jnp.float32),
                   jax.ShapeDtypeStruct((N_PAD, 1), jnp.float32)),
    )(x_pad, W1, c0, c1)


def _tc_layer(pa, pb, g, dinv, b, W):
    def body(pa_ref, pb_ref, g_ref, dinv_ref, b_ref, w_ref, o_ref):
        dinv = dinv_ref[...]
        xin = jax.nn.relu(dinv * (pa_ref[...] + pb_ref[...] + g_ref[...])
                          + b_ref[...])
        h = jnp.dot(xin, w_ref[...],
                    preferred_element_type=jnp.float32,
                    precision=lax.Precision.HIGHEST)
        o_ref[...] = h * dinv

    return pl.pallas_call(
        body,
        out_shape=jax.ShapeDtypeStruct((N_PAD, W.shape[1]), jnp.float32),
    )(pa, pb, g, dinv, b, W)


def _tc_final(pa, pb, g, dinv, b):
    def body(pa_ref, pb_ref, g_ref, dinv_ref, b_ref, o_ref):
        o_ref[...] = (dinv_ref[...] * (pa_ref[...] + pb_ref[...] + g_ref[...])
                      + b_ref[...])

    return pl.pallas_call(
        body,
        out_shape=jax.ShapeDtypeStruct((N_PAD, g.shape[1]), jnp.float32),
    )(pa, pb, g, dinv, b)


def kernel(x, edge_index, W1, b1, W2, b2, W3, b3):
    src3 = edge_index[0].astype(jnp.int32).reshape(NW, NWIN, WIN)
    dst3 = edge_index[1].astype(jnp.int32).reshape(NW, NWIN, WIN)
    x_pad = jnp.pad(x, ((0, N_PAD - N_NODES), (0, 0)))

    cnt = _sc_degree(dst3)
    c0 = cnt[0][:, None]
    c1 = cnt[1][:, None]

    g1, dinv = _tc_first(x_pad, W1, c0, c1)
    p1 = _sc_propagate(g1, src3, dst3, 128)
    g2 = _tc_layer(p1[0], p1[1], g1, dinv, b1[None, :], W2)
    p2 = _sc_propagate(g2, src3, dst3, 128)

    d3p = 128
    W3p = jnp.pad(W3, ((0, 0), (0, d3p - W3.shape[1])))
    b3p = jnp.pad(b3, (0, d3p - b3.shape[0]))
    g3 = _tc_layer(p2[0], p2[1], g2, dinv, b2[None, :], W3p)
    p3 = _sc_propagate(g3, src3, dst3, d3p)
    out = _tc_final(p3[0], p3[1], g3, dinv, b3p[None, :])
    return out[:N_NODES, :W3.shape[1]]


# speedup vs baseline: 27.3716x; 2.5694x over previous
import functools

import jax
import jax.numpy as jnp
from jax import lax
from jax.experimental import pallas as pl
from jax.experimental.pallas import tpu as pltpu
from jax.experimental.pallas import tpu_sc as plsc

N_NODES = 10000
N_PAD = 10240
N_EDGES = 320000
NC = 2
NS = 16
NW = NC * NS
WIN = 128
NWIN = 80
EPW = NWIN * WIN
RPT = N_PAD // NS
ZROWS = 128

_MESH = plsc.VectorSubcoreMesh(core_axis_name="c", subcore_axis_name="s")


def _sc_degree(dst3):
    @functools.partial(
        pl.kernel,
        out_type=jax.ShapeDtypeStruct((NC, N_PAD), jnp.float32),
        mesh=_MESH,
        scratch_types=[
            pltpu.VMEM_SHARED((N_PAD,), jnp.float32),
            pltpu.VMEM((NWIN, WIN), jnp.int32),
            pltpu.VMEM((1, WIN), jnp.float32),
            pltpu.VMEM((RPT,), jnp.float32),
            pltpu.SemaphoreType.DMA,
        ],
    )
    def k(dst_hbm, out_hbm, acc, idx_v, ones_v, z_v, sem):
        c = lax.axis_index("c")
        s = lax.axis_index("s")
        wid = c * NS + s
        pltpu.async_copy(dst_hbm.at[wid], idx_v, sem)

        @pl.loop(0, WIN, step=16)
        def _(i):
            ones_v[0, pl.ds(i, 16)] = jnp.ones((16,), jnp.float32)

        @pl.loop(0, RPT, step=16)
        def _(i):
            z_v[pl.ds(i, 16)] = jnp.zeros((16,), jnp.float32)

        pltpu.sync_copy(z_v, acc.at[pl.ds(s * RPT, RPT)])
        plsc.subcore_barrier()
        pltpu.make_async_copy(dst_hbm.at[wid], idx_v, sem).wait()

        @pl.loop(0, NWIN)
        def _(w):
            pltpu.sync_copy(ones_v.at[0], acc.at[idx_v.at[w]], add=True)

        plsc.subcore_barrier()
        pltpu.sync_copy(acc.at[pl.ds(s * RPT, RPT)],
                        out_hbm.at[c, pl.ds(s * RPT, RPT)])

    return k(dst3)


def _sc_propagate(g_pad, src3, dst3, d_ch):
    @functools.partial(
        pl.kernel,
        out_type=jax.ShapeDtypeStruct((NC, N_PAD, d_ch), jnp.float32),
        mesh=_MESH,
        scratch_types=[
            pltpu.VMEM_SHARED((N_PAD, d_ch), jnp.float32),
            pltpu.VMEM((NWIN, WIN), jnp.int32),
            pltpu.VMEM((1, WIN), jnp.int32),
            pltpu.VMEM((1, WIN), jnp.int32),
            pltpu.VMEM((WIN, d_ch), jnp.float32),
            pltpu.VMEM((WIN, d_ch), jnp.float32),
            pltpu.SemaphoreType.DMA,
            pltpu.SemaphoreType.DMA,
            pltpu.SemaphoreType.DMA,
            pltpu.SemaphoreType.DMA,
            pltpu.SemaphoreType.DMA,
        ],
    )
    def k(g_hbm, src_hbm, dst_hbm, out_hbm, acc, sidx, didx_a, didx_b,
          rows_a, rows_b, sem_i, sem_a, sem_b, sem_da, sem_db):
        c = lax.axis_index("c")
        s = lax.axis_index("s")
        wid = c * NS + s
        pltpu.async_copy(src_hbm.at[wid], sidx, sem_i)

        @pl.loop(0, WIN)
        def _(r):
            @pl.loop(0, d_ch, step=16)
            def _(k16):
                rows_a[r, pl.ds(k16, 16)] = jnp.zeros((16,), jnp.float32)

        @pl.loop(0, RPT, step=WIN)
        def _(r0):
            pltpu.sync_copy(rows_a, acc.at[pl.ds(s * RPT + r0, WIN)])

        pltpu.make_async_copy(src_hbm.at[wid], sidx, sem_i).wait()
        pltpu.async_copy(dst_hbm.at[wid, pl.ds(0, 1)], didx_a, sem_da)
        pltpu.async_copy(dst_hbm.at[wid, pl.ds(1, 1)], didx_b, sem_db)
        pltpu.async_copy(g_hbm.at[sidx.at[0]], rows_a, sem_a)
        pltpu.async_copy(g_hbm.at[sidx.at[1]], rows_b, sem_b)
        plsc.subcore_barrier()

        @pl.loop(0, NWIN, step=2)
        def _(w):
            pltpu.make_async_copy(g_hbm.at[sidx.at[w]], rows_a, sem_a).wait()
            pltpu.make_async_copy(dst_hbm.at[wid, pl.ds(w, 1)], didx_a,
                                  sem_da).wait()
            pltpu.sync_copy(rows_a, acc.at[didx_a.at[0]], add=True)

            @pl.when(w + 2 < NWIN)
            def _():
                pltpu.async_copy(dst_hbm.at[wid, pl.ds(w + 2, 1)], didx_a,
                                 sem_da)
                pltpu.async_copy(g_hbm.at[sidx.at[w + 2]], rows_a, sem_a)

            pltpu.make_async_copy(g_hbm.at[sidx.at[w + 1]], rows_b,
                                  sem_b).wait()
            pltpu.make_async_copy(dst_hbm.at[wid, pl.ds(w + 1, 1)], didx_b,
                                  sem_db).wait()
            pltpu.sync_copy(rows_b, acc.at[didx_b.at[0]], add=True)

            @pl.when(w + 3 < NWIN)
            def _():
                pltpu.async_copy(dst_hbm.at[wid, pl.ds(w + 3, 1)], didx_b,
                                 sem_db)
                pltpu.async_copy(g_hbm.at[sidx.at[w + 3]], rows_b, sem_b)

        plsc.subcore_barrier()
        pltpu.sync_copy(acc.at[pl.ds(s * RPT, RPT)],
                        out_hbm.at[c, pl.ds(s * RPT, RPT)])

    return k(g_pad, src3, dst3)


def _tc_first(x_pad, W1, c0, c1):
    def body(x_ref, w_ref, c0_ref, c1_ref, g_ref, dinv_ref):
        dinv = lax.rsqrt(c0_ref[...] + c1_ref[...] + 1.0)
        dinv_ref[...] = dinv
        h = jnp.dot(x_ref[...], w_ref[...],
                    preferred_element_type=jnp.float32,
                    precision=lax.Precision.HIGHEST)
        g_ref[...] = h * dinv

    return pl.pallas_call(
        body,
        out_shape=(jax.ShapeDtypeStruct((N_PAD, W1.shape[1]), jnp.float32),
                   jax.ShapeDtypeStruct((N_PAD, 1), jnp.float32)),
    )(x_pad, W1, c0, c1)


def _tc_layer(pa, pb, g, dinv, b, W):
    def body(pa_ref, pb_ref, g_ref, dinv_ref, b_ref, w_ref, o_ref):
        dinv = dinv_ref[...]
        xin = jax.nn.relu(dinv * (pa_ref[...] + pb_ref[...] + g_ref[...])
                          + b_ref[...])
        h = jnp.dot(xin, w_ref[...],
                    preferred_element_type=jnp.float32,
                    precision=lax.Precision.HIGHEST)
        o_ref[...] = h * dinv

    return pl.pallas_call(
        body,
        out_shape=jax.ShapeDtypeStruct((N_PAD, W.shape[1]), jnp.float32),
    )(pa, pb, g, dinv, b, W)


def _tc_final(pa, pb, g, dinv, b):
    def body(pa_ref, pb_ref, g_ref, dinv_ref, b_ref, o_ref):
        o_ref[...] = (dinv_ref[...] * (pa_ref[...] + pb_ref[...] + g_ref[...])
                      + b_ref[...])

    return pl.pallas_call(
        body,
        out_shape=jax.ShapeDtypeStruct((N_PAD, g.shape[1]), jnp.float32),
    )(pa, pb, g, dinv, b)


def _edge_windows(edge_row):
    e = edge_row.astype(jnp.int32).reshape(NW, N_EDGES // NW)
    n_fill = EPW - N_EDGES // NW
    fill = (N_NODES
            + (jnp.arange(NW * n_fill, dtype=jnp.int32) % (N_PAD - N_NODES))
            ).reshape(NW, n_fill)
    return jnp.concatenate([e, fill], axis=1).reshape(NW, NWIN, WIN)


def kernel(x, edge_index, W1, b1, W2, b2, W3, b3):
    src3 = _edge_windows(edge_index[0])
    dst3 = _edge_windows(edge_index[1])
    x_pad = jnp.pad(x, ((0, N_PAD - N_NODES), (0, 0)))

    cnt = _sc_degree(dst3)
    c0 = cnt[0][:, None]
    c1 = cnt[1][:, None]

    g1, dinv = _tc_first(x_pad, W1, c0, c1)
    p1 = _sc_propagate(g1, src3, dst3, 128)
    g2 = _tc_layer(p1[0], p1[1], g1, dinv, b1[None, :], W2)
    p2 = _sc_propagate(g2, src3, dst3, 128)

    d3p = 128
    W3p = jnp.pad(W3, ((0, 0), (0, d3p - W3.shape[1])))
    b3p = jnp.pad(b3, (0, d3p - b3.shape[0]))
    g3 = _tc_layer(p2[0], p2[1], g2, dinv, b2[None, :], W3p)
    p3 = _sc_propagate(g3, src3, dst3, d3p)
    out = _tc_final(p3[0], p3[1], g3, dinv, b3p[None, :])
    return out[:N_NODES, :W3.shape[1]]


# speedup vs baseline: 29.3894x; 1.0737x over previous
import functools

import jax
import jax.numpy as jnp
from jax import lax
from jax.experimental import pallas as pl
from jax.experimental.pallas import tpu as pltpu
from jax.experimental.pallas import tpu_sc as plsc

N_NODES = 10000
N_PAD = 10240
N_EDGES = 320000
IN_CH = 128
NC = 2
NS = 16
NW = NC * NS
WIN = 128
NWIN = 80
EPW = NWIN * WIN
RPT = N_PAD // NS

_MESH = plsc.VectorSubcoreMesh(core_axis_name="c", subcore_axis_name="s")


def _sc_degree(dst3):
    @functools.partial(
        pl.kernel,
        out_type=jax.ShapeDtypeStruct((NC, N_PAD), jnp.float32),
        mesh=_MESH,
        scratch_types=[
            pltpu.VMEM_SHARED((N_PAD,), jnp.float32),
            pltpu.VMEM((NWIN, WIN), jnp.int32),
            pltpu.VMEM((1, WIN), jnp.float32),
            pltpu.VMEM((RPT,), jnp.float32),
            pltpu.SemaphoreType.DMA,
        ],
    )
    def k(dst_hbm, out_hbm, acc, idx_v, ones_v, z_v, sem):
        c = lax.axis_index("c")
        s = lax.axis_index("s")
        wid = c * NS + s
        pltpu.async_copy(dst_hbm.at[wid], idx_v, sem)

        @pl.loop(0, WIN, step=16)
        def _(i):
            ones_v[0, pl.ds(i, 16)] = jnp.ones((16,), jnp.float32)

        @pl.loop(0, RPT, step=16)
        def _(i):
            z_v[pl.ds(i, 16)] = jnp.zeros((16,), jnp.float32)

        pltpu.sync_copy(z_v, acc.at[pl.ds(s * RPT, RPT)])
        plsc.subcore_barrier()
        pltpu.make_async_copy(dst_hbm.at[wid], idx_v, sem).wait()

        @pl.loop(0, NWIN)
        def _(w):
            pltpu.sync_copy(ones_v.at[0], acc.at[idx_v.at[w]], add=True)

        plsc.subcore_barrier()
        pltpu.sync_copy(acc.at[pl.ds(s * RPT, RPT)],
                        out_hbm.at[c, pl.ds(s * RPT, RPT)])

    return k(dst3)


def _sc_propagate(g_pad, src3, dst3, d_ch):
    @functools.partial(
        pl.kernel,
        out_type=jax.ShapeDtypeStruct((NC, N_PAD, d_ch), jnp.float32),
        mesh=_MESH,
        scratch_types=[
            pltpu.VMEM_SHARED((N_PAD, d_ch), jnp.float32),
            pltpu.VMEM((NWIN, WIN), jnp.int32),
            pltpu.VMEM((1, WIN), jnp.int32),
            pltpu.VMEM((1, WIN), jnp.int32),
            pltpu.VMEM((WIN, d_ch), jnp.float32),
            pltpu.VMEM((WIN, d_ch), jnp.float32),
            pltpu.SemaphoreType.DMA,
            pltpu.SemaphoreType.DMA,
            pltpu.SemaphoreType.DMA,
            pltpu.SemaphoreType.DMA,
            pltpu.SemaphoreType.DMA,
        ],
    )
    def k(g_hbm, src_hbm, dst_hbm, out_hbm, acc, sidx, didx_a, didx_b,
          rows_a, rows_b, sem_i, sem_a, sem_b, sem_da, sem_db):
        c = lax.axis_index("c")
        s = lax.axis_index("s")
        wid = c * NS + s
        pltpu.async_copy(src_hbm.at[wid], sidx, sem_i)

        @pl.loop(0, WIN)
        def _(r):
            @pl.loop(0, d_ch, step=16)
            def _(k16):
                rows_a[r, pl.ds(k16, 16)] = jnp.zeros((16,), jnp.float32)

        @pl.loop(0, RPT, step=WIN)
        def _(r0):
            pltpu.sync_copy(rows_a, acc.at[pl.ds(s * RPT + r0, WIN)])

        pltpu.make_async_copy(src_hbm.at[wid], sidx, sem_i).wait()
        pltpu.async_copy(dst_hbm.at[wid, pl.ds(0, 1)], didx_a, sem_da)
        pltpu.async_copy(dst_hbm.at[wid, pl.ds(1, 1)], didx_b, sem_db)
        pltpu.async_copy(g_hbm.at[sidx.at[0]], rows_a, sem_a)
        pltpu.async_copy(g_hbm.at[sidx.at[1]], rows_b, sem_b)
        plsc.subcore_barrier()

        @pl.loop(0, NWIN, step=2)
        def _(w):
            pltpu.make_async_copy(g_hbm.at[sidx.at[w]], rows_a, sem_a).wait()
            pltpu.make_async_copy(dst_hbm.at[wid, pl.ds(w, 1)], didx_a,
                                  sem_da).wait()
            pltpu.sync_copy(rows_a, acc.at[didx_a.at[0]], add=True)

            @pl.when(w + 2 < NWIN)
            def _():
                pltpu.async_copy(dst_hbm.at[wid, pl.ds(w + 2, 1)], didx_a,
                                 sem_da)
                pltpu.async_copy(g_hbm.at[sidx.at[w + 2]], rows_a, sem_a)

            pltpu.make_async_copy(g_hbm.at[sidx.at[w + 1]], rows_b,
                                  sem_b).wait()
            pltpu.make_async_copy(dst_hbm.at[wid, pl.ds(w + 1, 1)], didx_b,
                                  sem_db).wait()
            pltpu.sync_copy(rows_b, acc.at[didx_b.at[0]], add=True)

            @pl.when(w + 3 < NWIN)
            def _():
                pltpu.async_copy(dst_hbm.at[wid, pl.ds(w + 3, 1)], didx_b,
                                 sem_db)
                pltpu.async_copy(g_hbm.at[sidx.at[w + 3]], rows_b, sem_b)

        plsc.subcore_barrier()
        pltpu.sync_copy(acc.at[pl.ds(s * RPT, RPT)],
                        out_hbm.at[c, pl.ds(s * RPT, RPT)])

    return k(g_pad, src3, dst3)


def _tc_first(x, W1, cnt):
    def body(x_ref, w_ref, cnt_ref, g_ref, dinv_ref):
        c = cnt_ref[...]
        deg_row = c[0:1, :] + c[1:2, :] + 1.0
        dinv_col = jnp.reshape(lax.rsqrt(deg_row), (N_PAD, 1))
        dinv_ref[...] = dinv_col
        h = jnp.dot(x_ref[...], w_ref[...],
                    preferred_element_type=jnp.float32,
                    precision=lax.Precision.HIGHEST)
        g_ref[0:N_NODES, :] = h * dinv_col[0:N_NODES, :]

    return pl.pallas_call(
        body,
        out_shape=(jax.ShapeDtypeStruct((N_PAD, W1.shape[1]), jnp.float32),
                   jax.ShapeDtypeStruct((N_PAD, 1), jnp.float32)),
    )(x, W1, cnt)


def _tc_layer(p, g, dinv, b, W):
    d_out = W.shape[1]

    def body(p_ref, g_ref, dinv_ref, b_ref, w_ref, o_ref):
        dinv = dinv_ref[...]
        xin = jax.nn.relu(dinv * (p_ref[0] + p_ref[1] + g_ref[...])
                          + b_ref[...])
        h = jnp.dot(xin, w_ref[...],
                    preferred_element_type=jnp.float32,
                    precision=lax.Precision.HIGHEST)
        if d_out == g_ref.shape[1]:
            o_ref[...] = h * dinv
        else:
            o_ref[:, 0:d_out] = h * dinv

    return pl.pallas_call(
        body,
        out_shape=jax.ShapeDtypeStruct((N_PAD, g.shape[1]), jnp.float32),
    )(p, g, dinv, b, W)


def _tc_final(p, g, dinv, b, d_out):
    def body(p_ref, g_ref, dinv_ref, b_ref, o_ref):
        v = dinv_ref[...] * (p_ref[0] + p_ref[1] + g_ref[...])
        o_ref[...] = v[0:N_NODES, 0:d_out] + b_ref[...]

    return pl.pallas_call(
        body,
        out_shape=jax.ShapeDtypeStruct((N_NODES, d_out), jnp.float32),
    )(p, g, dinv, b)


def _edge_windows(edge_row):
    e = edge_row.astype(jnp.int32).reshape(NW, N_EDGES // NW)
    n_fill = EPW - N_EDGES // NW
    fill = (N_NODES
            + (jnp.arange(NW * n_fill, dtype=jnp.int32) % (N_PAD - N_NODES))
            ).reshape(NW, n_fill)
    return jnp.concatenate([e, fill], axis=1).reshape(NW, NWIN, WIN)


def kernel(x, edge_index, W1, b1, W2, b2, W3, b3):
    src3 = _edge_windows(edge_index[0])
    dst3 = _edge_windows(edge_index[1])

    cnt = _sc_degree(dst3)
    g1, dinv = _tc_first(x, W1, cnt)
    p1 = _sc_propagate(g1, src3, dst3, 128)
    g2 = _tc_layer(p1, g1, dinv, b1[None, :], W2)
    p2 = _sc_propagate(g2, src3, dst3, 128)
    g3 = _tc_layer(p2, g2, dinv, b2[None, :], W3)
    p3 = _sc_propagate(g3, src3, dst3, 128)
    return _tc_final(p3, g3, dinv, b3[None, :], W3.shape[1])
